# baseline (speedup 1.0000x reference)
import functools

import jax
import jax.numpy as jnp
from jax import lax
from jax.experimental import pallas as pl
from jax.experimental.pallas import tpu as pltpu
from jax.experimental.pallas import tpu_sc as plsc

N = 10000
E = 320000
D = 128

NC = 2
NS = 16
NW = NC * NS

K = 128
RB = 512
B_FAST = 34
B_SLOW = 6
E_PER_W = B_FAST * RB
E_FAST = NS * B_FAST * RB
E_SLOW = NS * B_SLOW * RB
E_PAD = E_FAST + E_SLOW
N_PAD = 10240
STRIPE = N_PAD // NS
LANES = 16


def _sc_aggregate(feat_pad, src_r, dst_r, zblk, ones, zvec):
    mesh = plsc.VectorSubcoreMesh(core_axis_name="c", subcore_axis_name="s")

    @functools.partial(
        pl.kernel,
        mesh=mesh,
        out_type=[
            jax.ShapeDtypeStruct((NC, N_PAD, D), jnp.float32),
            jax.ShapeDtypeStruct((NC, N_PAD), jnp.float32),
        ],
        scratch_types=[
            pltpu.VMEM((RB,), jnp.int32),
            pltpu.VMEM((RB,), jnp.int32),
            pltpu.VMEM((RB,), jnp.int32),
            pltpu.VMEM((RB,), jnp.int32),
            pltpu.VMEM((K, D), jnp.float32),
            pltpu.VMEM((K, D), jnp.float32),
            pltpu.VMEM((RB,), jnp.float32),
            pltpu.VMEM((STRIPE,), jnp.float32),
            pltpu.VMEM_SHARED((N_PAD, D), jnp.float32),
            pltpu.VMEM_SHARED((N_PAD,), jnp.float32),
            pltpu.SemaphoreType.DMA,
            pltpu.SemaphoreType.DMA,
            pltpu.SemaphoreType.DMA,
            pltpu.SemaphoreType.DMA,
            pltpu.SemaphoreType.DMA,
            pltpu.SemaphoreType.DMA,
        ],
    )
    def agg(feat_hbm, src_hbm, dst_hbm, zblk_hbm, ones_hbm, zvec_hbm,
            q_hbm, pdeg_hbm,
            sidx0, sidx1, didx0, didx1, ra_v, rb_v, ones_v, zvec_v,
            accum_sh, deg_sh,
            sga, sgb, ssa, ssb, six, sdg):
        c = lax.axis_index("c")
        s = lax.axis_index("s")
        wid = c * NS + s

        pltpu.sync_copy(zblk_hbm, ra_v)
        pltpu.sync_copy(ones_hbm, ones_v)
        pltpu.sync_copy(zvec_hbm, zvec_v)
        base = s * STRIPE
        for b in range(STRIPE // K):
            pltpu.sync_copy(ra_v, accum_sh.at[pl.ds(base + b * K, K)])
        pltpu.sync_copy(zvec_v, deg_sh.at[pl.ds(base, STRIPE)])
        plsc.subcore_barrier()

        sidx = (sidx0, sidx1)
        didx = (didx0, didx1)
        bufs = (ra_v, rb_v)
        gsem = (sga, sgb)
        ssem = (ssa, ssb)

        def load_batch(tb, slot, sync):
            off = pl.ds(tb * RB, RB)
            if sync:
                pltpu.sync_copy(src_hbm.at[wid, off], sidx[slot])
                pltpu.sync_copy(dst_hbm.at[wid, off], didx[slot])
                return None
            return (pltpu.async_copy(src_hbm.at[wid, off], sidx[slot], six),
                    pltpu.async_copy(dst_hbm.at[wid, off], didx[slot], six))

        def run_batch(slot, prefetch_tb):
            dcp = pltpu.async_copy(ones_v, deg_sh.at[didx[slot]], sdg, add=True)
            pf = (load_batch(prefetch_tb, 1 - slot, sync=False)
                  if prefetch_tb is not None else None)

            def gather(k, w):
                return pltpu.async_copy(
                    feat_hbm.at[sidx[slot].at[pl.ds(k * K, K)]],
                    bufs[w], gsem[w])

            def scat(k, w):
                return pltpu.async_copy(
                    bufs[w],
                    accum_sh.at[didx[slot].at[pl.ds(k * K, K)]],
                    ssem[w], add=True)

            g0 = gather(0, 0)
            g1 = gather(1, 1)
            g0.wait()
            s0 = scat(0, 0)
            g1.wait()
            s1 = scat(1, 1)
            s0.wait()
            g2 = gather(2, 0)
            g2.wait()
            s2 = scat(2, 0)
            s1.wait()
            g3 = gather(3, 1)
            g3.wait()
            s3 = scat(3, 1)
            s2.wait()
            s3.wait()
            dcp.wait()
            return pf

        def batch_pair(t2, carry):
            tb0 = 2 * t2
            load_batch(tb0, 0, sync=True)
            pf = run_batch(0, tb0 + 1)
            for cp in pf:
                cp.wait()
            run_batch(1, None)
            return carry

        npairs = jnp.where(c == 0, B_FAST // 2, B_SLOW // 2)
        lax.fori_loop(0, npairs, batch_pair, 0)
        plsc.subcore_barrier()

        for b in range(STRIPE // K):
            pltpu.sync_copy(accum_sh.at[pl.ds(base + b * K, K)],
                            q_hbm.at[c, pl.ds(base + b * K, K)])
        pltpu.sync_copy(deg_sh.at[pl.ds(base, STRIPE)],
                        pdeg_hbm.at[c, pl.ds(base, STRIPE)])

    return agg(feat_pad, src_r, dst_r, zblk, ones, zvec)


def kernel(feat, edge_index, eps):
    src = edge_index[0]
    dst = edge_index[1]
    pad = E_PAD - E
    src_p = jnp.concatenate([src, jnp.zeros((pad,), jnp.int32)])
    dst_p = jnp.concatenate([dst, jnp.full((pad,), N, jnp.int32)])

    def split_rows(x, fill):
        fast = x[:E_FAST].reshape(NS, E_PER_W)
        slow = x[E_FAST:].reshape(NS, B_SLOW * RB)
        slow = jnp.concatenate(
            [slow, jnp.full((NS, (B_FAST - B_SLOW) * RB), fill, jnp.int32)],
            axis=1)
        return jnp.concatenate([fast, slow], axis=0)

    src_r = split_rows(src_p, 0)
    dst_r = split_rows(dst_p, N)
    feat_pad = jnp.concatenate(
        [feat, jnp.zeros((N_PAD - N, D), jnp.float32)], axis=0)

    q, pdeg = _sc_aggregate(feat_pad, src_r, dst_r,
                            jnp.zeros((K, D), jnp.float32),
                            jnp.ones((RB,), jnp.float32),
                            jnp.zeros((STRIPE,), jnp.float32))

    deg = pdeg[0] + pdeg[1]
    invb = jnp.broadcast_to(
        (1.0 / jnp.maximum(deg, 1.0))[:, None], (N_PAD, D))

    BLK = 512
    eps2 = jnp.reshape(eps, (1, 1)).astype(jnp.float32)

    def combine(eps_ref, feat_ref, q0_ref, q1_ref, inv_ref, out_ref):
        out_ref[...] = ((1.0 + eps_ref[0, 0]) * feat_ref[...]
                        + (q0_ref[0] + q1_ref[0]) * inv_ref[...])

    out = pl.pallas_call(
        combine,
        grid=(N_PAD // BLK,),
        in_specs=[
            pl.BlockSpec((1, 1), lambda i: (0, 0)),
            pl.BlockSpec((BLK, D), lambda i: (i, 0)),
            pl.BlockSpec((1, BLK, D), lambda i: (0, i, 0)),
            pl.BlockSpec((1, BLK, D), lambda i: (1, i, 0)),
            pl.BlockSpec((BLK, D), lambda i: (i, 0)),
        ],
        out_specs=pl.BlockSpec((BLK, D), lambda i: (i, 0)),
        out_shape=jax.ShapeDtypeStruct((N_PAD, D), jnp.float32),
    )(eps2, feat_pad, q, q, invb)
    return out[:N]


# speedup vs baseline: 1.0081x; 1.0081x over previous
import functools

import jax
import jax.numpy as jnp
from jax import lax
from jax.experimental import pallas as pl
from jax.experimental.pallas import tpu as pltpu
from jax.experimental.pallas import tpu_sc as plsc

N = 10000
E = 320000
D = 128

NC = 2
NS = 16
NW = NC * NS

K = 128
RB = 512
B_FAST = 32
B_SLOW = 8
E_PER_W = B_FAST * RB
E_FAST = NS * B_FAST * RB
E_SLOW = NS * B_SLOW * RB
E_PAD = E_FAST + E_SLOW
N_PAD = 10240
STRIPE = N_PAD // NS
LANES = 16


def _sc_aggregate(feat_pad, src_r, dst_r, zblk, ones, zvec):
    mesh = plsc.VectorSubcoreMesh(core_axis_name="c", subcore_axis_name="s")

    @functools.partial(
        pl.kernel,
        mesh=mesh,
        out_type=[
            jax.ShapeDtypeStruct((NC, N_PAD, D), jnp.float32),
            jax.ShapeDtypeStruct((NC, N_PAD), jnp.float32),
        ],
        scratch_types=[
            pltpu.VMEM((RB,), jnp.int32),
            pltpu.VMEM((RB,), jnp.int32),
            pltpu.VMEM((RB,), jnp.int32),
            pltpu.VMEM((RB,), jnp.int32),
            pltpu.VMEM((K, D), jnp.float32),
            pltpu.VMEM((K, D), jnp.float32),
            pltpu.VMEM((RB,), jnp.float32),
            pltpu.VMEM((STRIPE,), jnp.float32),
            pltpu.VMEM_SHARED((N_PAD, D), jnp.float32),
            pltpu.VMEM_SHARED((N_PAD,), jnp.float32),
            pltpu.SemaphoreType.DMA,
            pltpu.SemaphoreType.DMA,
            pltpu.SemaphoreType.DMA,
            pltpu.SemaphoreType.DMA,
            pltpu.SemaphoreType.DMA,
            pltpu.SemaphoreType.DMA,
        ],
    )
    def agg(feat_hbm, src_hbm, dst_hbm, zblk_hbm, ones_hbm, zvec_hbm,
            q_hbm, pdeg_hbm,
            sidx0, sidx1, didx0, didx1, ra_v, rb_v, ones_v, zvec_v,
            accum_sh, deg_sh,
            sga, sgb, ssa, ssb, six, sdg):
        c = lax.axis_index("c")
        s = lax.axis_index("s")
        wid = c * NS + s

        pltpu.sync_copy(zblk_hbm, ra_v)
        pltpu.sync_copy(ones_hbm, ones_v)
        pltpu.sync_copy(zvec_hbm, zvec_v)
        base = s * STRIPE
        for b in range(STRIPE // K):
            pltpu.sync_copy(ra_v, accum_sh.at[pl.ds(base + b * K, K)])
        pltpu.sync_copy(zvec_v, deg_sh.at[pl.ds(base, STRIPE)])
        plsc.subcore_barrier()

        sidx = (sidx0, sidx1)
        didx = (didx0, didx1)
        bufs = (ra_v, rb_v)
        gsem = (sga, sgb)
        ssem = (ssa, ssb)

        def load_batch(tb, slot, sync):
            off = pl.ds(tb * RB, RB)
            if sync:
                pltpu.sync_copy(src_hbm.at[wid, off], sidx[slot])
                pltpu.sync_copy(dst_hbm.at[wid, off], didx[slot])
                return None
            return (pltpu.async_copy(src_hbm.at[wid, off], sidx[slot], six),
                    pltpu.async_copy(dst_hbm.at[wid, off], didx[slot], six))

        def run_batch(slot, prefetch_tb):
            dcp = pltpu.async_copy(ones_v, deg_sh.at[didx[slot]], sdg, add=True)
            pf = (load_batch(prefetch_tb, 1 - slot, sync=False)
                  if prefetch_tb is not None else None)

            def gather(k, w):
                return pltpu.async_copy(
                    feat_hbm.at[sidx[slot].at[pl.ds(k * K, K)]],
                    bufs[w], gsem[w])

            def scat(k, w):
                return pltpu.async_copy(
                    bufs[w],
                    accum_sh.at[didx[slot].at[pl.ds(k * K, K)]],
                    ssem[w], add=True)

            g0 = gather(0, 0)
            g1 = gather(1, 1)
            g0.wait()
            s0 = scat(0, 0)
            g1.wait()
            s1 = scat(1, 1)
            s0.wait()
            g2 = gather(2, 0)
            g2.wait()
            s2 = scat(2, 0)
            s1.wait()
            g3 = gather(3, 1)
            g3.wait()
            s3 = scat(3, 1)
            s2.wait()
            s3.wait()
            dcp.wait()
            return pf

        def batch_pair(t2, carry):
            tb0 = 2 * t2
            load_batch(tb0, 0, sync=True)
            pf = run_batch(0, tb0 + 1)
            for cp in pf:
                cp.wait()
            run_batch(1, None)
            return carry

        npairs = jnp.where(c == 0, B_FAST // 2, B_SLOW // 2)
        lax.fori_loop(0, npairs, batch_pair, 0)
        plsc.subcore_barrier()

        for b in range(STRIPE // K):
            pltpu.sync_copy(accum_sh.at[pl.ds(base + b * K, K)],
                            q_hbm.at[c, pl.ds(base + b * K, K)])
        pltpu.sync_copy(deg_sh.at[pl.ds(base, STRIPE)],
                        pdeg_hbm.at[c, pl.ds(base, STRIPE)])

    return agg(feat_pad, src_r, dst_r, zblk, ones, zvec)


def kernel(feat, edge_index, eps):
    src = edge_index[0]
    dst = edge_index[1]
    pad = E_PAD - E
    src_p = jnp.concatenate([src, jnp.zeros((pad,), jnp.int32)])
    dst_p = jnp.concatenate([dst, jnp.full((pad,), N, jnp.int32)])

    def split_rows(x, fill):
        fast = x[:E_FAST].reshape(NS, E_PER_W)
        slow = x[E_FAST:].reshape(NS, B_SLOW * RB)
        slow = jnp.concatenate(
            [slow, jnp.full((NS, (B_FAST - B_SLOW) * RB), fill, jnp.int32)],
            axis=1)
        return jnp.concatenate([fast, slow], axis=0)

    src_r = split_rows(src_p, 0)
    dst_r = split_rows(dst_p, N)
    feat_pad = jnp.concatenate(
        [feat, jnp.zeros((N_PAD - N, D), jnp.float32)], axis=0)

    q, pdeg = _sc_aggregate(feat_pad, src_r, dst_r,
                            jnp.zeros((K, D), jnp.float32),
                            jnp.ones((RB,), jnp.float32),
                            jnp.zeros((STRIPE,), jnp.float32))

    deg = pdeg[0] + pdeg[1]
    invb = jnp.broadcast_to(
        (1.0 / jnp.maximum(deg, 1.0))[:, None], (N_PAD, D))

    BLK = 512
    eps2 = jnp.reshape(eps, (1, 1)).astype(jnp.float32)

    def combine(eps_ref, feat_ref, q0_ref, q1_ref, inv_ref, out_ref):
        out_ref[...] = ((1.0 + eps_ref[0, 0]) * feat_ref[...]
                        + (q0_ref[0] + q1_ref[0]) * inv_ref[...])

    out = pl.pallas_call(
        combine,
        grid=(N_PAD // BLK,),
        in_specs=[
            pl.BlockSpec((1, 1), lambda i: (0, 0)),
            pl.BlockSpec((BLK, D), lambda i: (i, 0)),
            pl.BlockSpec((1, BLK, D), lambda i: (0, i, 0)),
            pl.BlockSpec((1, BLK, D), lambda i: (1, i, 0)),
            pl.BlockSpec((BLK, D), lambda i: (i, 0)),
        ],
        out_specs=pl.BlockSpec((BLK, D), lambda i: (i, 0)),
        out_shape=jax.ShapeDtypeStruct((N_PAD, D), jnp.float32),
    )(eps2, feat_pad, q, q, invb)
    return out[:N]


# speedup vs baseline: 1.1712x; 1.1618x over previous
import functools

import jax
import jax.numpy as jnp
from jax import lax
from jax.experimental import pallas as pl
from jax.experimental.pallas import tpu as pltpu
from jax.experimental.pallas import tpu_sc as plsc

N = 10000
E = 320000
D = 128

NC = 2
NS = 16
NW = NC * NS

K = 128
RB = 512
B_FAST = 30
B_SLOW = 10
E_PER_W = B_FAST * RB
E_FAST = NS * B_FAST * RB
E_SLOW = NS * B_SLOW * RB
E_PAD = E_FAST + E_SLOW
N_PAD = 10240
STRIPE = N_PAD // NS
LANES = 16


def _sc_aggregate(feat_pad, src_r, dst_r, zblk, ones, zvec):
    mesh = plsc.VectorSubcoreMesh(core_axis_name="c", subcore_axis_name="s")

    @functools.partial(
        pl.kernel,
        mesh=mesh,
        out_type=[
            jax.ShapeDtypeStruct((NC, N_PAD, D), jnp.float32),
            jax.ShapeDtypeStruct((NC, N_PAD), jnp.float32),
        ],
        scratch_types=[
            pltpu.VMEM((RB,), jnp.int32),
            pltpu.VMEM((RB,), jnp.int32),
            pltpu.VMEM((RB,), jnp.int32),
            pltpu.VMEM((RB,), jnp.int32),
            pltpu.VMEM((K, D), jnp.float32),
            pltpu.VMEM((K, D), jnp.float32),
            pltpu.VMEM((RB,), jnp.float32),
            pltpu.VMEM((STRIPE,), jnp.float32),
            pltpu.VMEM_SHARED((N_PAD, D), jnp.float32),
            pltpu.VMEM_SHARED((N_PAD,), jnp.float32),
            pltpu.SemaphoreType.DMA,
            pltpu.SemaphoreType.DMA,
            pltpu.SemaphoreType.DMA,
            pltpu.SemaphoreType.DMA,
            pltpu.SemaphoreType.DMA,
            pltpu.SemaphoreType.DMA,
        ],
    )
    def agg(feat_hbm, src_hbm, dst_hbm, zblk_hbm, ones_hbm, zvec_hbm,
            q_hbm, pdeg_hbm,
            sidx0, sidx1, didx0, didx1, ra_v, rb_v, ones_v, zvec_v,
            accum_sh, deg_sh,
            sga, sgb, ssa, ssb, six, sdg):
        c = lax.axis_index("c")
        s = lax.axis_index("s")
        wid = c * NS + s

        pltpu.sync_copy(zblk_hbm, ra_v)
        pltpu.sync_copy(ones_hbm, ones_v)
        pltpu.sync_copy(zvec_hbm, zvec_v)
        base = s * STRIPE
        for b in range(STRIPE // K):
            pltpu.sync_copy(ra_v, accum_sh.at[pl.ds(base + b * K, K)])
        pltpu.sync_copy(zvec_v, deg_sh.at[pl.ds(base, STRIPE)])
        plsc.subcore_barrier()

        sidx = (sidx0, sidx1)
        didx = (didx0, didx1)
        bufs = (ra_v, rb_v)
        gsem = (sga, sgb)
        ssem = (ssa, ssb)

        def load_batch(tb, slot, sync):
            off = pl.ds(tb * RB, RB)
            if sync:
                pltpu.sync_copy(src_hbm.at[wid, off], sidx[slot])
                pltpu.sync_copy(dst_hbm.at[wid, off], didx[slot])
                return None
            return (pltpu.async_copy(src_hbm.at[wid, off], sidx[slot], six),
                    pltpu.async_copy(dst_hbm.at[wid, off], didx[slot], six))

        def run_batch(slot, prefetch_tb):
            dcp = pltpu.async_copy(ones_v, deg_sh.at[didx[slot]], sdg, add=True)
            pf = (load_batch(prefetch_tb, 1 - slot, sync=False)
                  if prefetch_tb is not None else None)

            def gather(k, w):
                return pltpu.async_copy(
                    feat_hbm.at[sidx[slot].at[pl.ds(k * K, K)]],
                    bufs[w], gsem[w])

            def scat(k, w):
                return pltpu.async_copy(
                    bufs[w],
                    accum_sh.at[didx[slot].at[pl.ds(k * K, K)]],
                    ssem[w], add=True)

            g0 = gather(0, 0)
            g1 = gather(1, 1)
            g0.wait()
            s0 = scat(0, 0)
            g1.wait()
            s1 = scat(1, 1)
            s0.wait()
            g2 = gather(2, 0)
            g2.wait()
            s2 = scat(2, 0)
            s1.wait()
            g3 = gather(3, 1)
            g3.wait()
            s3 = scat(3, 1)
            s2.wait()
            s3.wait()
            dcp.wait()
            return pf

        def batch_pair(t2, carry):
            tb0 = 2 * t2
            load_batch(tb0, 0, sync=True)
            pf = run_batch(0, tb0 + 1)
            for cp in pf:
                cp.wait()
            run_batch(1, None)
            return carry

        npairs = jnp.where(c == 0, B_FAST // 2, B_SLOW // 2)
        lax.fori_loop(0, npairs, batch_pair, 0)
        plsc.subcore_barrier()

        for b in range(STRIPE // K):
            pltpu.sync_copy(accum_sh.at[pl.ds(base + b * K, K)],
                            q_hbm.at[c, pl.ds(base + b * K, K)])
        pltpu.sync_copy(deg_sh.at[pl.ds(base, STRIPE)],
                        pdeg_hbm.at[c, pl.ds(base, STRIPE)])

    return agg(feat_pad, src_r, dst_r, zblk, ones, zvec)


def kernel(feat, edge_index, eps):
    src = edge_index[0]
    dst = edge_index[1]
    pad = E_PAD - E
    src_p = jnp.concatenate([src, jnp.zeros((pad,), jnp.int32)])
    dst_p = jnp.concatenate([dst, jnp.full((pad,), N, jnp.int32)])

    def split_rows(x, fill):
        fast = x[:E_FAST].reshape(NS, E_PER_W)
        slow = x[E_FAST:].reshape(NS, B_SLOW * RB)
        slow = jnp.concatenate(
            [slow, jnp.full((NS, (B_FAST - B_SLOW) * RB), fill, jnp.int32)],
            axis=1)
        return jnp.concatenate([fast, slow], axis=0)

    src_r = split_rows(src_p, 0)
    dst_r = split_rows(dst_p, N)
    feat_pad = jnp.concatenate(
        [feat, jnp.zeros((N_PAD - N, D), jnp.float32)], axis=0)

    q, pdeg = _sc_aggregate(feat_pad, src_r, dst_r,
                            jnp.zeros((K, D), jnp.float32),
                            jnp.ones((RB,), jnp.float32),
                            jnp.zeros((STRIPE,), jnp.float32))

    deg = pdeg[0] + pdeg[1]
    invb = jnp.broadcast_to(
        (1.0 / jnp.maximum(deg, 1.0))[:, None], (N_PAD, D))

    BLK = 512
    eps2 = jnp.reshape(eps, (1, 1)).astype(jnp.float32)

    def combine(eps_ref, feat_ref, q0_ref, q1_ref, inv_ref, out_ref):
        out_ref[...] = ((1.0 + eps_ref[0, 0]) * feat_ref[...]
                        + (q0_ref[0] + q1_ref[0]) * inv_ref[...])

    out = pl.pallas_call(
        combine,
        grid=(N_PAD // BLK,),
        in_specs=[
            pl.BlockSpec((1, 1), lambda i: (0, 0)),
            pl.BlockSpec((BLK, D), lambda i: (i, 0)),
            pl.BlockSpec((1, BLK, D), lambda i: (0, i, 0)),
            pl.BlockSpec((1, BLK, D), lambda i: (1, i, 0)),
            pl.BlockSpec((BLK, D), lambda i: (i, 0)),
        ],
        out_specs=pl.BlockSpec((BLK, D), lambda i: (i, 0)),
        out_shape=jax.ShapeDtypeStruct((N_PAD, D), jnp.float32),
    )(eps2, feat_pad, q, q, invb)
    return out[:N]
